# Initial kernel scaffold; baseline (speedup 1.0000x reference)
import functools

import jax
import jax.numpy as jnp
from jax import lax
from jax.experimental import pallas as pl
from jax.experimental.pallas import tpu as pltpu
from jax.experimental.pallas import tpu_sc as plsc

VOCAB = 1000000
DIM = 64
NEG = 5
BATCH = 16384

NC, NS, L = 2, 16, 16
NW = NC * NS
B_PER_W = BATCH // NW
C = 128
N_CHUNKS = B_PER_W // C
G = C // L


def _splat(v):
    return jnp.full((L,), v, jnp.int32)


def _sc_body(data_hbm, emb0_hbm, emb1_hbm, out_hbm,
             data_v, w_idx, c_idx, n_idx, w_rows, c_rows, n_rows, out_v, sem):
    wid = lax.axis_index("c") * NS + lax.axis_index("s")
    base = wid * B_PER_W
    iota = lax.iota(jnp.int32, L)

    def chunk_body(ch, _):
        cbase = base + ch * C
        pltpu.sync_copy(data_hbm.at[pl.ds(cbase, C)], data_v)

        for g in range(G):
            rows = g * L + iota
            w_idx[pl.ds(g * L, L)] = plsc.load_gather(data_v, [rows, _splat(0)])
            c_idx[pl.ds(g * L, L)] = plsc.load_gather(data_v, [rows, _splat(1)])
            for k in range(NEG):
                n_idx[k, pl.ds(g * L, L)] = plsc.load_gather(
                    data_v, [rows, _splat(2 + k)])

        cps = [pltpu.async_copy(emb0_hbm.at[w_idx], w_rows, sem),
               pltpu.async_copy(emb1_hbm.at[c_idx], c_rows, sem)]
        for k in range(NEG):
            cps.append(pltpu.async_copy(emb1_hbm.at[n_idx.at[k]],
                                        n_rows.at[k], sem))
        for cp in cps:
            cp.wait()

        def group_body(g, _):
            rows = g * L + iota
            accp = jnp.zeros((L,), jnp.float32)
            accn = [jnp.zeros((L,), jnp.float32) for _ in range(NEG)]
            for d in range(DIM):
                cold = _splat(d)
                wv = plsc.load_gather(w_rows, [rows, cold])
                cv = plsc.load_gather(c_rows, [rows, cold])
                accp = accp + wv * cv
                for k in range(NEG):
                    nv = plsc.load_gather(n_rows, [_splat(k), rows, cold])
                    accn[k] = accn[k] + wv * nv
            orow = ch * C + rows
            plsc.store_scatter(out_v, [orow, _splat(0)], accp)
            for k in range(NEG):
                plsc.store_scatter(out_v, [orow, _splat(k + 1)], accn[k])
            return 0

        lax.fori_loop(0, G, group_body, 0)
        return 0

    lax.fori_loop(0, N_CHUNKS, chunk_body, 0)
    pltpu.sync_copy(out_v, out_hbm.at[pl.ds(base, B_PER_W)])


def _sc_ips(data, emb0, emb1):
    mesh = plsc.VectorSubcoreMesh(core_axis_name="c", subcore_axis_name="s")
    return pl.kernel(
        _sc_body,
        out_type=jax.ShapeDtypeStruct((BATCH, 8), jnp.float32),
        mesh=mesh,
        scratch_types=[
            pltpu.VMEM((C, 12), jnp.int32),
            pltpu.VMEM((C,), jnp.int32),
            pltpu.VMEM((C,), jnp.int32),
            pltpu.VMEM((NEG, C), jnp.int32),
            pltpu.VMEM((C, DIM), jnp.float32),
            pltpu.VMEM((C, DIM), jnp.float32),
            pltpu.VMEM((NEG, C, DIM), jnp.float32),
            pltpu.VMEM((B_PER_W, 8), jnp.float32),
            pltpu.SemaphoreType.DMA,
        ],
    )(data, emb0, emb1)


def _tc_loss_body(ips_ref, data_ref, out_ref):
    ips = ips_ref[...]
    data = data_ref[...]
    pos = ips[:, 0:1]
    negs = ips[:, 1:1 + NEG]
    mask = data[:, 2 + NEG:].astype(jnp.float32)
    pos_l = jnp.sum(-jax.nn.log_sigmoid(jnp.clip(pos, -10.0, 10.0)))
    neg_l = jnp.sum(-jax.nn.log_sigmoid(jnp.clip(-negs, -10.0, 10.0)) * mask)
    out_ref[0, 0] = pos_l + neg_l


def _tc_loss(ips, data):
    return pl.pallas_call(
        _tc_loss_body,
        out_shape=jax.ShapeDtypeStruct((1, 1), jnp.float32),
    )(ips, data)


def kernel(data, emb0, emb1):
    ips = _sc_ips(data, emb0, emb1)
    return _tc_loss(ips, data)[0, 0]


# speedup vs baseline: 1.5546x; 1.5546x over previous
import functools

import jax
import jax.numpy as jnp
from jax import lax
from jax.experimental import pallas as pl
from jax.experimental.pallas import tpu as pltpu
from jax.experimental.pallas import tpu_sc as plsc

VOCAB = 1000000
DIM = 64
NEG = 5
BATCH = 16384

NC, NS, L = 2, 16, 16
NW = NC * NS
B_PER_W = BATCH // NW
C = 128
N_CHUNKS = B_PER_W // C
G = C // L


def _splat(v):
    return jnp.full((L,), v, jnp.int32)


def _sc_body(data_hbm, emb0_hbm, emb1_hbm, out_hbm,
             data_v, w_idx, c_idx, n_idx, w_rows, c_rows, n_rows, out_v, sem):
    wid = lax.axis_index("c") * NS + lax.axis_index("s")
    base = wid * B_PER_W
    iota = lax.iota(jnp.int32, L)

    def chunk_body(ch, _):
        cbase = base + ch * C
        pltpu.sync_copy(data_hbm.at[pl.ds(cbase, C)], data_v)

        for g in range(G):
            rows = g * L + iota
            w_idx[pl.ds(g * L, L)] = plsc.load_gather(data_v, [rows, _splat(0)])
            c_idx[pl.ds(g * L, L)] = plsc.load_gather(data_v, [rows, _splat(1)])
            for k in range(NEG):
                n_idx[k, pl.ds(g * L, L)] = plsc.load_gather(
                    data_v, [rows, _splat(2 + k)])

        cps = [pltpu.async_copy(emb0_hbm.at[w_idx], w_rows, sem),
               pltpu.async_copy(emb1_hbm.at[c_idx], c_rows, sem)]
        for k in range(NEG):
            cps.append(pltpu.async_copy(emb1_hbm.at[n_idx.at[k]],
                                        n_rows.at[k], sem))
        for cp in cps:
            cp.wait()

        def group_body(g, _):
            rows = g * L + iota
            accp = jnp.zeros((L,), jnp.float32)
            accn = [jnp.zeros((L,), jnp.float32) for _ in range(NEG)]
            for d in range(DIM):
                cold = _splat(d)
                wv = plsc.load_gather(w_rows, [rows, cold])
                cv = plsc.load_gather(c_rows, [rows, cold])
                accp = accp + wv * cv
                for k in range(NEG):
                    nv = plsc.load_gather(n_rows, [_splat(k), rows, cold])
                    accn[k] = accn[k] + wv * nv
            orow = ch * C + rows
            plsc.store_scatter(out_v, [orow, _splat(0)], accp)
            for k in range(NEG):
                plsc.store_scatter(out_v, [orow, _splat(k + 1)], accn[k])
            return 0

        lax.fori_loop(0, G, group_body, 0)
        return 0

    lax.fori_loop(0, N_CHUNKS, chunk_body, 0)
    pltpu.sync_copy(out_v, out_hbm.at[pl.ds(base, B_PER_W)])


def _sc_ips(data, emb0, emb1):
    mesh = plsc.VectorSubcoreMesh(core_axis_name="c", subcore_axis_name="s")
    return pl.kernel(
        _sc_body,
        out_type=jax.ShapeDtypeStruct((BATCH, 8), jnp.float32),
        mesh=mesh,
        compiler_params=pltpu.CompilerParams(
            needs_layout_passes=False, use_tc_tiling_on_sc=False),
        scratch_types=[
            pltpu.VMEM((C, 12), jnp.int32),
            pltpu.VMEM((C,), jnp.int32),
            pltpu.VMEM((C,), jnp.int32),
            pltpu.VMEM((NEG, C), jnp.int32),
            pltpu.VMEM((C, DIM), jnp.float32),
            pltpu.VMEM((C, DIM), jnp.float32),
            pltpu.VMEM((NEG, C, DIM), jnp.float32),
            pltpu.VMEM((B_PER_W, 8), jnp.float32),
            pltpu.SemaphoreType.DMA,
        ],
    )(data, emb0, emb1)


def _tc_loss_body(ips_ref, data_ref, out_ref):
    ips = ips_ref[...]
    data = data_ref[...]
    pos = ips[:, 0:1]
    negs = ips[:, 1:1 + NEG]
    mask = data[:, 2 + NEG:].astype(jnp.float32)
    pos_l = jnp.sum(-jax.nn.log_sigmoid(jnp.clip(pos, -10.0, 10.0)))
    neg_l = jnp.sum(-jax.nn.log_sigmoid(jnp.clip(-negs, -10.0, 10.0)) * mask)
    out_ref[...] = (pos_l + neg_l).reshape(1, 1)


def _tc_loss(ips, data):
    return pl.pallas_call(
        _tc_loss_body,
        out_shape=jax.ShapeDtypeStruct((1, 1), jnp.float32),
    )(ips, data)


def kernel(data, emb0, emb1):
    ips = _sc_ips(data, emb0, emb1)
    return _tc_loss(ips, data)[0, 0]
